# baseline (speedup 1.0000x reference)
import jax
import jax.numpy as jnp
from jax.experimental import pallas as pl
from jax.experimental.pallas import tpu as pltpu

N_E = 1024
E_DIM = 64
BETA = 0.25
B = 32
T = 16
BT = B * T
NCOL = N_E + 1


def _vq_kernel(z_ref, w_ref, zq_ref, loss_ref, ind_ref, v_ref, dscr):
    z3 = z_ref[...]
    z2d = z3.reshape(BT, E_DIM)
    w = w_ref[...]

    z2 = jnp.sum(z2d * z2d, axis=1, keepdims=True)
    ww = w * w
    w2r = jax.lax.dot_general(
        jnp.ones((1, E_DIM), jnp.float32), ww,
        (((1,), (1,)), ((), ())), preferred_element_type=jnp.float32)
    zw = jax.lax.dot_general(
        z2d, w, (((1,), (1,)), ((), ())),
        preferred_element_type=jnp.float32)
    d = (z2 + w2r) - 2.0 * zw
    dscr[...] = d.reshape(B, T, NCOL)

    col = jax.lax.broadcasted_iota(jnp.int32, (B, NCOL), 1)
    eps = 1e-06 / N_E

    d0 = dscr[:, 0, :]
    mn = jnp.min(d0, axis=1, keepdims=True)
    ind = jnp.min(jnp.where(d0 == mn, col, NCOL), axis=1, keepdims=True)
    ind = jnp.minimum(ind, N_E - 1)
    oh0 = col == ind
    dsel = jnp.sum(jnp.where(oh0, d0, 0.0), axis=1, keepdims=True)
    wsel = jax.lax.dot_general(
        jnp.where(oh0, 1.0, 0.0), w, (((1,), (0,)), ((), ())),
        preferred_element_type=jnp.float32)
    zt = z_ref[:, 0, :]
    zq_ref[:, 0, :] = zt + (wsel - zt)
    lacc = jnp.maximum((dsel - d0) + eps, 0.0)

    minv = ind
    maxv = ind
    ind_cols = [ind]
    for t in range(1, T):
        dt = dscr[:, t, :]
        indn = jnp.minimum(ind + 1, N_E - 1)
        ohh = col == ind
        ohn = col == indn
        here = jnp.sum(jnp.where(ohh, dt, 0.0), axis=1, keepdims=True)
        nxt = jnp.sum(jnp.where(ohn, dt, 0.0), axis=1, keepdims=True)
        keep = here <= nxt
        ind = jnp.where(keep, ind, indn)
        dsel = jnp.where(keep, here, nxt)
        wh = jax.lax.dot_general(
            jnp.where(ohh, 1.0, 0.0), w, (((1,), (0,)), ((), ())),
            preferred_element_type=jnp.float32)
        wn = jax.lax.dot_general(
            jnp.where(ohn, 1.0, 0.0), w, (((1,), (0,)), ((), ())),
            preferred_element_type=jnp.float32)
        wsel = jnp.where(keep, wh, wn)
        zt = z_ref[:, t, :]
        zq_ref[:, t, :] = zt + (wsel - zt)
        lacc = lacc + jnp.maximum((dsel - dt) + eps, 0.0)
        ind_cols.append(ind)
        minv = jnp.minimum(minv, ind)
        maxv = jnp.maximum(maxv, ind)

    ind_ref[...] = jnp.concatenate(ind_cols, axis=1)
    lc = jnp.sum(jnp.sum(lacc, axis=1, keepdims=True), axis=0,
                 keepdims=True) / float(BT * NCOL)
    loss_ref[...] = BETA * lc + lc
    v_ref[...] = jnp.max(maxv - minv, axis=0, keepdims=True)


def kernel(z, W):
    zq, loss, ind, v = pl.pallas_call(
        _vq_kernel,
        out_shape=[
            jax.ShapeDtypeStruct((B, T, E_DIM), jnp.float32),
            jax.ShapeDtypeStruct((1, 1), jnp.float32),
            jax.ShapeDtypeStruct((B, T), jnp.int32),
            jax.ShapeDtypeStruct((1, 1), jnp.int32),
        ],
        scratch_shapes=[pltpu.VMEM((B, T, NCOL), jnp.float32)],
    )(z, W)
    return (zq, loss.reshape(()), ind, v.reshape(()))
